# Initial kernel scaffold; baseline (speedup 1.0000x reference)
import jax
import jax.numpy as jnp
from jax.experimental import pallas as pl


def kernel(n_feats, r_feats, edge_index, W_O_w, W_O_b, W_I_w, W_I_b, W_S_w, W_S_b, W_R_w, W_R_b):
    raise NotImplementedError("write your pallas kernel here")


# speedup vs baseline: 10.7443x; 10.7443x over previous
import functools

import jax
import jax.numpy as jnp
from jax import lax
from jax.experimental import pallas as pl
from jax.experimental.pallas import tpu as pltpu
from jax.experimental.pallas import tpu_sc as plsc

_NC = 2
_NS = 16
_CHUNK = 80


@functools.lru_cache(maxsize=None)
def _make_sc_segment_sums(N, D, E):
    NS, NC, C = _NS, _NC, _CHUNK
    EPW = E // NS
    NCH = EPW // C
    NBLK = 5
    BCH = NCH // NBLK
    SROW_T = 10
    ROWS_T = N // SROW_T
    NW_FULL = ROWS_T // C
    W_TAIL = ROWS_T - NW_FULL * C
    DEG_T = 2000

    mesh = plsc.VectorSubcoreMesh(core_axis_name="c", subcore_axis_name="s")

    @functools.partial(
        pl.kernel,
        out_type=(
            jax.ShapeDtypeStruct((NC, N, D), jnp.float32),
            jax.ShapeDtypeStruct((N,), jnp.float32),
            jax.ShapeDtypeStruct((N,), jnp.float32),
        ),
        mesh=mesh,
        scratch_types=[
            pltpu.VMEM((BCH, C), jnp.int32),
            pltpu.VMEM((BCH, C), jnp.int32),
            pltpu.VMEM((C, D), jnp.float32),
            pltpu.VMEM((C, D), jnp.float32),
            pltpu.VMEM((C,), jnp.float32),
            pltpu.VMEM((DEG_T,), jnp.float32),
            pltpu.VMEM_SHARED((N, D), jnp.float32),
            pltpu.VMEM_SHARED((N,), jnp.float32),
            pltpu.SemaphoreType.DMA,
            pltpu.SemaphoreType.DMA,
        ],
    )
    def sc_kernel(nf_hbm, edges_hbm, s_out, deg_f_out, deg_r_out,
                  gidx, sidx, rows_a, rows_b, ones_v, dstage,
                  s_acc, deg_acc, sem_a, sem_b):
        c = lax.axis_index("c")
        s = lax.axis_index("s")

        zero16 = jnp.zeros((16,), jnp.float32)
        one16 = jnp.ones((16,), jnp.float32)
        for j in range(C // 16):
            ones_v[pl.ds(j * 16, 16)] = one16

        def _zrow(i, carry):
            for j in range(D // 16):
                rows_a[i, pl.ds(j * 16, 16)] = zero16
            return carry

        lax.fori_loop(0, C, _zrow, 0)

        def _zdeg(i, carry):
            dstage[pl.ds(i * 16, 16)] = zero16
            return carry

        lax.fori_loop(0, DEG_T // 16, _zdeg, 0)

        @pl.when(s < SROW_T)
        def _():
            for k in range(NW_FULL):
                pltpu.sync_copy(rows_a, s_acc.at[pl.ds(s * ROWS_T + k * C, C)])
            if W_TAIL:
                pltpu.sync_copy(
                    rows_a.at[pl.ds(0, W_TAIL)],
                    s_acc.at[pl.ds(s * ROWS_T + NW_FULL * C, W_TAIL)])

        @pl.when(s < N // DEG_T)
        def _():
            pltpu.sync_copy(dstage, deg_acc.at[pl.ds(s * DEG_T, DEG_T)])

        plsc.subcore_barrier()

        g = c
        r = 1 - c

        def _gather(j, buf, sem):
            pltpu.async_copy(nf_hbm.at[gidx.at[j]], buf, sem)

        def _wait(j, buf, sem):
            pltpu.make_async_copy(nf_hbm.at[gidx.at[j]], buf, sem).wait()

        def _scat(j, buf):
            pltpu.sync_copy(buf, s_acc.at[sidx.at[j]], add=True)
            pltpu.sync_copy(ones_v, deg_acc.at[sidx.at[j]], add=True)

        def _block(b, carry):
            pltpu.sync_copy(edges_hbm.at[g, s, b], gidx)
            pltpu.sync_copy(edges_hbm.at[r, s, b], sidx)
            _gather(0, rows_a, sem_a)

            def _body(jj, carry2):
                j0 = 2 * jj
                j1 = j0 + 1
                _gather(j1, rows_b, sem_b)
                _wait(j0, rows_a, sem_a)
                _scat(j0, rows_a)
                _gather(jnp.minimum(j1 + 1, BCH - 1), rows_a, sem_a)
                _wait(j1, rows_b, sem_b)
                _scat(j1, rows_b)
                return carry2

            lax.fori_loop(0, BCH // 2, _body, 0)
            _wait(BCH - 1, rows_a, sem_a)
            return carry

        lax.fori_loop(0, NBLK, _block, 0)

        plsc.subcore_barrier()

        @pl.when(s < SROW_T)
        def _():
            for k in range(NW_FULL):
                lo = s * ROWS_T + k * C
                pltpu.sync_copy(s_acc.at[pl.ds(lo, C)], rows_a)
                pltpu.sync_copy(rows_a, s_out.at[c, pl.ds(lo, C)])
            if W_TAIL:
                lo = s * ROWS_T + NW_FULL * C
                pltpu.sync_copy(
                    s_acc.at[pl.ds(lo, W_TAIL)], rows_b.at[pl.ds(0, W_TAIL)])
                pltpu.sync_copy(
                    rows_b.at[pl.ds(0, W_TAIL)], s_out.at[c, pl.ds(lo, W_TAIL)])

        @pl.when(s < N // DEG_T)
        def _():
            pltpu.sync_copy(deg_acc.at[pl.ds(s * DEG_T, DEG_T)], dstage)

            @pl.when(c == 0)
            def _():
                pltpu.sync_copy(dstage, deg_f_out.at[pl.ds(s * DEG_T, DEG_T)])

            @pl.when(c == 1)
            def _():
                pltpu.sync_copy(dstage, deg_r_out.at[pl.ds(s * DEG_T, DEG_T)])

    return sc_kernel


@functools.lru_cache(maxsize=None)
def _make_tc_combine(N, D, RPAD):
    R = 400
    G = N // R
    dn = (((1,), (1,)), ((), ()))
    f32 = jnp.float32

    def body(nf, sf, sr, df, dr, rp, wo, wi, ws, wr, bo, bi, bs, br,
             out, rout):
        i = pl.program_id(0)
        rp_v = rp[...]
        rw_o = lax.dot_general(rp_v, wo[...], dn, preferred_element_type=f32)
        rw_i = lax.dot_general(rp_v, wi[...], dn, preferred_element_type=f32)
        rw_s = lax.dot_general(rp_v, ws[...], dn, preferred_element_type=f32)
        c_f = bo[...] - rw_o[1:2, :]
        c_r = bi[...] - rw_i[2:3, :]
        c_s = bs[...] - rw_s[2:3, :]
        df_v = df[...]
        dr_v = dr[...]
        a_f = sf[...] * (1.0 / jnp.maximum(df_v, 1.0))
        a_r = sr[...] * (1.0 / jnp.maximum(dr_v, 1.0))
        acc = lax.dot_general(a_f, wo[...], dn, preferred_element_type=f32)
        acc += lax.dot_general(a_r, wi[...], dn, preferred_element_type=f32)
        acc += lax.dot_general(nf[...], ws[...], dn, preferred_element_type=f32)
        ind_f = jnp.where(df_v > 0.0, 1.0, 0.0)
        ind_r = jnp.where(dr_v > 0.0, 1.0, 0.0)
        out[...] = acc + ind_f * c_f + ind_r * c_r + c_s

        @pl.when(i == 0)
        def _():
            rout[...] = (
                lax.dot_general(rp_v, wr[...], dn, preferred_element_type=f32)
                + br[...]
            )

    row_blk = pl.BlockSpec((R, D), lambda i: (i, 0))
    col_blk = pl.BlockSpec((R, 1), lambda i: (i, 0))
    full = lambda shape: pl.BlockSpec(shape, lambda i: (0,) * len(shape))

    return pl.pallas_call(
        body,
        grid=(G,),
        in_specs=[
            row_blk, row_blk, row_blk, col_blk, col_blk,
            full((RPAD, D)),
            full((D, D)), full((D, D)), full((D, D)), full((D, D)),
            full((1, D)), full((1, D)), full((1, D)), full((1, D)),
        ],
        out_specs=[row_blk, full((RPAD, D))],
        out_shape=(
            jax.ShapeDtypeStruct((N, D), f32),
            jax.ShapeDtypeStruct((RPAD, D), f32),
        ),
    )


def kernel(n_feats, r_feats, edge_index, W_O_w, W_O_b, W_I_w, W_I_b,
           W_S_w, W_S_b, W_R_w, W_R_b):
    N, D = n_feats.shape
    E = edge_index.shape[1]
    NR = r_feats.shape[0]
    RPAD = 8

    NCH = (E // _NS) // _CHUNK
    edges_r = edge_index.reshape(2, _NS, 5, NCH // 5, _CHUNK)
    S, deg_f, deg_r = _make_sc_segment_sums(N, D, E)(n_feats, edges_r)

    rp = jnp.zeros((RPAD, D), jnp.float32).at[:NR].set(r_feats)
    n_out, r_out = _make_tc_combine(N, D, RPAD)(
        n_feats,
        S[0], S[1],
        deg_f.reshape(N, 1), deg_r.reshape(N, 1),
        rp,
        W_O_w, W_I_w, W_S_w, W_R_w,
        W_O_b.reshape(1, D), W_I_b.reshape(1, D),
        W_S_b.reshape(1, D), W_R_b.reshape(1, D),
    )
    return n_out, r_out[:NR]


# speedup vs baseline: 11.0396x; 1.0275x over previous
import functools

import jax
import jax.numpy as jnp
from jax import lax
from jax.experimental import pallas as pl
from jax.experimental.pallas import tpu as pltpu
from jax.experimental.pallas import tpu_sc as plsc

_NC = 2
_NS = 16
_CHUNK = 80


@functools.lru_cache(maxsize=None)
def _make_sc_segment_sums(N, D, E):
    NS, NC, C = _NS, _NC, _CHUNK
    EPW = E // NS
    NCH = EPW // C
    NBLK = 5
    BCH = NCH // NBLK
    SROW_T = 10
    ROWS_T = N // SROW_T
    NW_FULL = ROWS_T // C
    W_TAIL = ROWS_T - NW_FULL * C
    DEG_T = 2000

    mesh = plsc.VectorSubcoreMesh(core_axis_name="c", subcore_axis_name="s")

    @functools.partial(
        pl.kernel,
        out_type=(
            jax.ShapeDtypeStruct((NC, N, D), jnp.float32),
            jax.ShapeDtypeStruct((N,), jnp.float32),
            jax.ShapeDtypeStruct((N,), jnp.float32),
        ),
        mesh=mesh,
        scratch_types=[
            pltpu.VMEM((BCH, C), jnp.int32),
            pltpu.VMEM((BCH, C), jnp.int32),
            pltpu.VMEM((C, D), jnp.float32),
            pltpu.VMEM((C, D), jnp.float32),
            pltpu.VMEM((C,), jnp.float32),
            pltpu.VMEM((DEG_T,), jnp.float32),
            pltpu.VMEM_SHARED((N, D), jnp.float32),
            pltpu.VMEM_SHARED((N,), jnp.float32),
            pltpu.SemaphoreType.DMA,
            pltpu.SemaphoreType.DMA,
            pltpu.SemaphoreType.DMA,
            pltpu.SemaphoreType.DMA,
            pltpu.SemaphoreType.DMA,
            pltpu.SemaphoreType.DMA,
        ],
    )
    def sc_kernel(nf_hbm, edges_hbm, s_out, deg_f_out, deg_r_out,
                  gidx, sidx, rows_a, rows_b, ones_v, dstage,
                  s_acc, deg_acc, gsem_a, gsem_b, ssem_a, ssem_b,
                  dsem_a, dsem_b):
        c = lax.axis_index("c")
        s = lax.axis_index("s")

        zero16 = jnp.zeros((16,), jnp.float32)
        one16 = jnp.ones((16,), jnp.float32)
        for j in range(C // 16):
            ones_v[pl.ds(j * 16, 16)] = one16

        def _zrow(i, carry):
            for j in range(D // 16):
                rows_a[i, pl.ds(j * 16, 16)] = zero16
            return carry

        lax.fori_loop(0, C, _zrow, 0)

        def _zdeg(i, carry):
            dstage[pl.ds(i * 16, 16)] = zero16
            return carry

        lax.fori_loop(0, DEG_T // 16, _zdeg, 0)

        @pl.when(s < SROW_T)
        def _():
            for k in range(NW_FULL):
                pltpu.sync_copy(rows_a, s_acc.at[pl.ds(s * ROWS_T + k * C, C)])
            if W_TAIL:
                pltpu.sync_copy(
                    rows_a.at[pl.ds(0, W_TAIL)],
                    s_acc.at[pl.ds(s * ROWS_T + NW_FULL * C, W_TAIL)])

        @pl.when(s < N // DEG_T)
        def _():
            pltpu.sync_copy(dstage, deg_acc.at[pl.ds(s * DEG_T, DEG_T)])

        plsc.subcore_barrier()

        g = c
        r = 1 - c

        def _gather(j, buf, sem):
            pltpu.async_copy(nf_hbm.at[gidx.at[j]], buf, sem)

        def _gwait(j, buf, sem):
            pltpu.make_async_copy(nf_hbm.at[gidx.at[j]], buf, sem).wait()

        def _scat(j, buf, sem, dsem):
            pltpu.async_copy(buf, s_acc.at[sidx.at[j]], sem, add=True)
            pltpu.async_copy(ones_v, deg_acc.at[sidx.at[j]], dsem, add=True)

        def _swait(j, buf, sem, dsem):
            pltpu.make_async_copy(buf, s_acc.at[sidx.at[j]], sem).wait()
            pltpu.make_async_copy(ones_v, deg_acc.at[sidx.at[j]], dsem).wait()

        def _block(b, carry):
            pltpu.sync_copy(edges_hbm.at[g, s, b], gidx)
            pltpu.sync_copy(edges_hbm.at[r, s, b], sidx)

            _gather(0, rows_a, gsem_a)
            _gwait(0, rows_a, gsem_a)
            _scat(0, rows_a, ssem_a, dsem_a)
            _gather(1, rows_b, gsem_b)

            def _body(jj, carry2):
                jo = 2 * jj + 1
                je = jo + 1
                _swait(jo - 1, rows_a, ssem_a, dsem_a)
                _gather(je, rows_a, gsem_a)
                _gwait(jo, rows_b, gsem_b)
                _scat(jo, rows_b, ssem_b, dsem_b)
                _swait(jo, rows_b, ssem_b, dsem_b)
                _gather(je + 1, rows_b, gsem_b)
                _gwait(je, rows_a, gsem_a)
                _scat(je, rows_a, ssem_a, dsem_a)
                return carry2

            lax.fori_loop(0, (BCH - 2) // 2, _body, 0)

            _swait(BCH - 2, rows_a, ssem_a, dsem_a)
            _gwait(BCH - 1, rows_b, gsem_b)
            _scat(BCH - 1, rows_b, ssem_b, dsem_b)
            _swait(BCH - 1, rows_b, ssem_b, dsem_b)
            return carry

        lax.fori_loop(0, NBLK, _block, 0)

        plsc.subcore_barrier()

        @pl.when(s < SROW_T)
        def _():
            for k in range(NW_FULL):
                lo = s * ROWS_T + k * C
                pltpu.sync_copy(s_acc.at[pl.ds(lo, C)], rows_a)
                pltpu.sync_copy(rows_a, s_out.at[c, pl.ds(lo, C)])
            if W_TAIL:
                lo = s * ROWS_T + NW_FULL * C
                pltpu.sync_copy(
                    s_acc.at[pl.ds(lo, W_TAIL)], rows_b.at[pl.ds(0, W_TAIL)])
                pltpu.sync_copy(
                    rows_b.at[pl.ds(0, W_TAIL)], s_out.at[c, pl.ds(lo, W_TAIL)])

        @pl.when(s < N // DEG_T)
        def _():
            pltpu.sync_copy(deg_acc.at[pl.ds(s * DEG_T, DEG_T)], dstage)

            @pl.when(c == 0)
            def _():
                pltpu.sync_copy(dstage, deg_f_out.at[pl.ds(s * DEG_T, DEG_T)])

            @pl.when(c == 1)
            def _():
                pltpu.sync_copy(dstage, deg_r_out.at[pl.ds(s * DEG_T, DEG_T)])

    return sc_kernel


@functools.lru_cache(maxsize=None)
def _make_tc_combine(N, D, RPAD):
    R = 400
    G = N // R
    dn = (((1,), (1,)), ((), ()))
    f32 = jnp.float32

    def body(nf, sf, sr, df, dr, rp, wo, wi, ws, wr, bo, bi, bs, br,
             out, rout):
        i = pl.program_id(0)
        rp_v = rp[...]
        rw_o = lax.dot_general(rp_v, wo[...], dn, preferred_element_type=f32)
        rw_i = lax.dot_general(rp_v, wi[...], dn, preferred_element_type=f32)
        rw_s = lax.dot_general(rp_v, ws[...], dn, preferred_element_type=f32)
        c_f = bo[...] - rw_o[1:2, :]
        c_r = bi[...] - rw_i[2:3, :]
        c_s = bs[...] - rw_s[2:3, :]
        df_v = df[...]
        dr_v = dr[...]
        a_f = sf[...] * (1.0 / jnp.maximum(df_v, 1.0))
        a_r = sr[...] * (1.0 / jnp.maximum(dr_v, 1.0))
        acc = lax.dot_general(a_f, wo[...], dn, preferred_element_type=f32)
        acc += lax.dot_general(a_r, wi[...], dn, preferred_element_type=f32)
        acc += lax.dot_general(nf[...], ws[...], dn, preferred_element_type=f32)
        ind_f = jnp.where(df_v > 0.0, 1.0, 0.0)
        ind_r = jnp.where(dr_v > 0.0, 1.0, 0.0)
        out[...] = acc + ind_f * c_f + ind_r * c_r + c_s

        @pl.when(i == 0)
        def _():
            rout[...] = (
                lax.dot_general(rp_v, wr[...], dn, preferred_element_type=f32)
                + br[...]
            )

    row_blk = pl.BlockSpec((R, D), lambda i: (i, 0))
    col_blk = pl.BlockSpec((R, 1), lambda i: (i, 0))
    full = lambda shape: pl.BlockSpec(shape, lambda i: (0,) * len(shape))

    return pl.pallas_call(
        body,
        grid=(G,),
        in_specs=[
            row_blk, row_blk, row_blk, col_blk, col_blk,
            full((RPAD, D)),
            full((D, D)), full((D, D)), full((D, D)), full((D, D)),
            full((1, D)), full((1, D)), full((1, D)), full((1, D)),
        ],
        out_specs=[row_blk, full((RPAD, D))],
        out_shape=(
            jax.ShapeDtypeStruct((N, D), f32),
            jax.ShapeDtypeStruct((RPAD, D), f32),
        ),
    )


def kernel(n_feats, r_feats, edge_index, W_O_w, W_O_b, W_I_w, W_I_b,
           W_S_w, W_S_b, W_R_w, W_R_b):
    N, D = n_feats.shape
    E = edge_index.shape[1]
    NR = r_feats.shape[0]
    RPAD = 8

    NCH = (E // _NS) // _CHUNK
    edges_r = edge_index.reshape(2, _NS, 5, NCH // 5, _CHUNK)
    S, deg_f, deg_r = _make_sc_segment_sums(N, D, E)(n_feats, edges_r)

    rp = jnp.zeros((RPAD, D), jnp.float32).at[:NR].set(r_feats)
    n_out, r_out = _make_tc_combine(N, D, RPAD)(
        n_feats,
        S[0], S[1],
        deg_f.reshape(N, 1), deg_r.reshape(N, 1),
        rp,
        W_O_w, W_I_w, W_S_w, W_R_w,
        W_O_b.reshape(1, D), W_I_b.reshape(1, D),
        W_S_b.reshape(1, D), W_R_b.reshape(1, D),
    )
    return n_out, r_out[:NR]


# speedup vs baseline: 12.0613x; 1.0925x over previous
import functools

import jax
import jax.numpy as jnp
from jax import lax
from jax.experimental import pallas as pl
from jax.experimental.pallas import tpu as pltpu
from jax.experimental.pallas import tpu_sc as plsc

_NC = 2
_NS = 16
_CHUNK = 125


@functools.lru_cache(maxsize=None)
def _make_sc_segment_sums(N, D, E):
    NS, NC, C = _NS, _NC, _CHUNK
    EPW = E // NS
    NCH = EPW // C
    NBLK = 4
    BCH = NCH // NBLK
    SROW_T = 10
    ROWS_T = N // SROW_T
    WCH = 120
    NW_FULL = ROWS_T // WCH
    W_TAIL = ROWS_T - NW_FULL * WCH
    DEG_T = 2000

    mesh = plsc.VectorSubcoreMesh(core_axis_name="c", subcore_axis_name="s")

    @functools.partial(
        pl.kernel,
        out_type=(
            jax.ShapeDtypeStruct((NC, N, D), jnp.float32),
            jax.ShapeDtypeStruct((N,), jnp.float32),
            jax.ShapeDtypeStruct((N,), jnp.float32),
        ),
        mesh=mesh,
        scratch_types=[
            pltpu.VMEM((BCH, C), jnp.int32),
            pltpu.VMEM((BCH, C), jnp.int32),
            pltpu.VMEM((C, D), jnp.float32),
            pltpu.VMEM((C, D), jnp.float32),
            pltpu.VMEM((128,), jnp.float32),
            pltpu.VMEM((DEG_T,), jnp.float32),
            pltpu.VMEM_SHARED((N, D), jnp.float32),
            pltpu.VMEM_SHARED((N,), jnp.float32),
            pltpu.SemaphoreType.DMA,
            pltpu.SemaphoreType.DMA,
            pltpu.SemaphoreType.DMA,
            pltpu.SemaphoreType.DMA,
            pltpu.SemaphoreType.DMA,
            pltpu.SemaphoreType.DMA,
        ],
    )
    def sc_kernel(nf_hbm, edges_hbm, s_out, deg_f_out, deg_r_out,
                  gidx, sidx, rows_a, rows_b, ones_v, dstage,
                  s_acc, deg_acc, gsem_a, gsem_b, ssem_a, ssem_b,
                  dsem_a, dsem_b):
        c = lax.axis_index("c")
        s = lax.axis_index("s")

        zero16 = jnp.zeros((16,), jnp.float32)
        one16 = jnp.ones((16,), jnp.float32)
        for j in range(128 // 16):
            ones_v[pl.ds(j * 16, 16)] = one16

        def _zrow(i, carry):
            for j in range(D // 16):
                rows_a[i, pl.ds(j * 16, 16)] = zero16
            return carry

        lax.fori_loop(0, C, _zrow, 0)

        def _zdeg(i, carry):
            dstage[pl.ds(i * 16, 16)] = zero16
            return carry

        lax.fori_loop(0, DEG_T // 16, _zdeg, 0)

        @pl.when(s < SROW_T)
        def _():
            for k in range(NW_FULL):
                pltpu.sync_copy(
                    rows_a.at[pl.ds(0, WCH)],
                    s_acc.at[pl.ds(s * ROWS_T + k * WCH, WCH)])
            if W_TAIL:
                pltpu.sync_copy(
                    rows_a.at[pl.ds(0, W_TAIL)],
                    s_acc.at[pl.ds(s * ROWS_T + NW_FULL * WCH, W_TAIL)])

        @pl.when(s < N // DEG_T)
        def _():
            pltpu.sync_copy(dstage, deg_acc.at[pl.ds(s * DEG_T, DEG_T)])

        plsc.subcore_barrier()

        g = c
        r = 1 - c

        def _gather(j, buf, sem):
            pltpu.async_copy(nf_hbm.at[gidx.at[j]], buf, sem)

        def _gwait(j, buf, sem):
            pltpu.make_async_copy(nf_hbm.at[gidx.at[j]], buf, sem).wait()

        def _scat(j, buf, sem, dsem):
            pltpu.async_copy(buf, s_acc.at[sidx.at[j]], sem, add=True)
            pltpu.async_copy(
                ones_v.at[pl.ds(0, C)], deg_acc.at[sidx.at[j]], dsem, add=True)

        def _swait(j, buf, sem, dsem):
            pltpu.make_async_copy(buf, s_acc.at[sidx.at[j]], sem).wait()
            pltpu.make_async_copy(
                ones_v.at[pl.ds(0, C)], deg_acc.at[sidx.at[j]], dsem).wait()

        def _block(b, carry):
            pltpu.sync_copy(edges_hbm.at[g, s, b], gidx)
            pltpu.sync_copy(edges_hbm.at[r, s, b], sidx)

            _gather(0, rows_a, gsem_a)
            _gwait(0, rows_a, gsem_a)
            _scat(0, rows_a, ssem_a, dsem_a)
            _gather(1, rows_b, gsem_b)

            def _body(jj, carry2):
                jo = 2 * jj + 1
                je = jo + 1
                _swait(jo - 1, rows_a, ssem_a, dsem_a)
                _gather(je, rows_a, gsem_a)
                _gwait(jo, rows_b, gsem_b)
                _scat(jo, rows_b, ssem_b, dsem_b)
                _swait(jo, rows_b, ssem_b, dsem_b)
                _gather(je + 1, rows_b, gsem_b)
                _gwait(je, rows_a, gsem_a)
                _scat(je, rows_a, ssem_a, dsem_a)
                return carry2

            lax.fori_loop(0, (BCH - 2) // 2, _body, 0)

            _swait(BCH - 2, rows_a, ssem_a, dsem_a)
            _gwait(BCH - 1, rows_b, gsem_b)
            _scat(BCH - 1, rows_b, ssem_b, dsem_b)
            _swait(BCH - 1, rows_b, ssem_b, dsem_b)
            return carry

        lax.fori_loop(0, NBLK, _block, 0)

        plsc.subcore_barrier()

        @pl.when(s < SROW_T)
        def _():
            for k in range(NW_FULL):
                lo = s * ROWS_T + k * WCH
                pltpu.sync_copy(s_acc.at[pl.ds(lo, WCH)], rows_a.at[pl.ds(0, WCH)])
                pltpu.sync_copy(rows_a.at[pl.ds(0, WCH)], s_out.at[c, pl.ds(lo, WCH)])
            if W_TAIL:
                lo = s * ROWS_T + NW_FULL * WCH
                pltpu.sync_copy(
                    s_acc.at[pl.ds(lo, W_TAIL)], rows_b.at[pl.ds(0, W_TAIL)])
                pltpu.sync_copy(
                    rows_b.at[pl.ds(0, W_TAIL)], s_out.at[c, pl.ds(lo, W_TAIL)])

        @pl.when(s < N // DEG_T)
        def _():
            pltpu.sync_copy(deg_acc.at[pl.ds(s * DEG_T, DEG_T)], dstage)

            @pl.when(c == 0)
            def _():
                pltpu.sync_copy(dstage, deg_f_out.at[pl.ds(s * DEG_T, DEG_T)])

            @pl.when(c == 1)
            def _():
                pltpu.sync_copy(dstage, deg_r_out.at[pl.ds(s * DEG_T, DEG_T)])

    return sc_kernel


@functools.lru_cache(maxsize=None)
def _make_tc_combine(N, D, RPAD):
    R = 400
    G = N // R
    dn = (((1,), (1,)), ((), ()))
    f32 = jnp.float32

    def body(nf, sf, sr, df, dr, rp, wo, wi, ws, wr, bo, bi, bs, br,
             out, rout):
        i = pl.program_id(0)
        rp_v = rp[...]
        rw_o = lax.dot_general(rp_v, wo[...], dn, preferred_element_type=f32)
        rw_i = lax.dot_general(rp_v, wi[...], dn, preferred_element_type=f32)
        rw_s = lax.dot_general(rp_v, ws[...], dn, preferred_element_type=f32)
        c_f = bo[...] - rw_o[1:2, :]
        c_r = bi[...] - rw_i[2:3, :]
        c_s = bs[...] - rw_s[2:3, :]
        df_v = df[...]
        dr_v = dr[...]
        a_f = sf[...] * (1.0 / jnp.maximum(df_v, 1.0))
        a_r = sr[...] * (1.0 / jnp.maximum(dr_v, 1.0))
        acc = lax.dot_general(a_f, wo[...], dn, preferred_element_type=f32)
        acc += lax.dot_general(a_r, wi[...], dn, preferred_element_type=f32)
        acc += lax.dot_general(nf[...], ws[...], dn, preferred_element_type=f32)
        ind_f = jnp.where(df_v > 0.0, 1.0, 0.0)
        ind_r = jnp.where(dr_v > 0.0, 1.0, 0.0)
        out[...] = acc + ind_f * c_f + ind_r * c_r + c_s

        @pl.when(i == 0)
        def _():
            rout[...] = (
                lax.dot_general(rp_v, wr[...], dn, preferred_element_type=f32)
                + br[...]
            )

    row_blk = pl.BlockSpec((R, D), lambda i: (i, 0))
    col_blk = pl.BlockSpec((R, 1), lambda i: (i, 0))
    full = lambda shape: pl.BlockSpec(shape, lambda i: (0,) * len(shape))

    return pl.pallas_call(
        body,
        grid=(G,),
        in_specs=[
            row_blk, row_blk, row_blk, col_blk, col_blk,
            full((RPAD, D)),
            full((D, D)), full((D, D)), full((D, D)), full((D, D)),
            full((1, D)), full((1, D)), full((1, D)), full((1, D)),
        ],
        out_specs=[row_blk, full((RPAD, D))],
        out_shape=(
            jax.ShapeDtypeStruct((N, D), f32),
            jax.ShapeDtypeStruct((RPAD, D), f32),
        ),
    )


def kernel(n_feats, r_feats, edge_index, W_O_w, W_O_b, W_I_w, W_I_b,
           W_S_w, W_S_b, W_R_w, W_R_b):
    N, D = n_feats.shape
    E = edge_index.shape[1]
    NR = r_feats.shape[0]
    RPAD = 8

    NCH = (E // _NS) // _CHUNK
    edges_r = edge_index.reshape(2, _NS, 4, NCH // 4, _CHUNK)
    S, deg_f, deg_r = _make_sc_segment_sums(N, D, E)(n_feats, edges_r)

    rp = jnp.zeros((RPAD, D), jnp.float32).at[:NR].set(r_feats)
    n_out, r_out = _make_tc_combine(N, D, RPAD)(
        n_feats,
        S[0], S[1],
        deg_f.reshape(N, 1), deg_r.reshape(N, 1),
        rp,
        W_O_w, W_I_w, W_S_w, W_R_w,
        W_O_b.reshape(1, D), W_I_b.reshape(1, D),
        W_S_b.reshape(1, D), W_R_b.reshape(1, D),
    )
    return n_out, r_out[:NR]


# speedup vs baseline: 12.5094x; 1.0372x over previous
import functools

import jax
import jax.numpy as jnp
from jax import lax
from jax.experimental import pallas as pl
from jax.experimental.pallas import tpu as pltpu
from jax.experimental.pallas import tpu_sc as plsc

_NC = 2
_NS = 16
_CHUNK = 100


@functools.lru_cache(maxsize=None)
def _make_sc_segment_sums(N, D, E):
    NS, NC, C = _NS, _NC, _CHUNK
    EPW = E // NS
    NCH = EPW // C
    NBLK = 8
    BCH = NCH // NBLK
    SROW_T = 10
    ROWS_T = N // SROW_T
    WCH = 96
    NW_FULL = ROWS_T // WCH
    W_TAIL = ROWS_T - NW_FULL * WCH
    DEG_T = 2000

    mesh = plsc.VectorSubcoreMesh(core_axis_name="c", subcore_axis_name="s")

    @functools.partial(
        pl.kernel,
        out_type=(
            jax.ShapeDtypeStruct((NC, N, D), jnp.float32),
            jax.ShapeDtypeStruct((N,), jnp.float32),
            jax.ShapeDtypeStruct((N,), jnp.float32),
        ),
        mesh=mesh,
        scratch_types=[
            pltpu.VMEM((BCH, C), jnp.int32),
            pltpu.VMEM((BCH, C), jnp.int32),
            pltpu.VMEM((C, D), jnp.float32),
            pltpu.VMEM((C, D), jnp.float32),
            pltpu.VMEM((C, D), jnp.float32),
            pltpu.VMEM((128,), jnp.float32),
            pltpu.VMEM((DEG_T,), jnp.float32),
            pltpu.VMEM_SHARED((N, D), jnp.float32),
            pltpu.VMEM_SHARED((N,), jnp.float32),
            [pltpu.SemaphoreType.DMA] * 3,
            [pltpu.SemaphoreType.DMA] * 3,
            [pltpu.SemaphoreType.DMA] * 3,
        ],
    )
    def sc_kernel(nf_hbm, edges_hbm, s_out, deg_f_out, deg_r_out,
                  gidx, sidx, rows_0, rows_1, rows_2, ones_v, dstage,
                  s_acc, deg_acc, gsems, ssems, dsems):
        rows = (rows_0, rows_1, rows_2)
        c = lax.axis_index("c")
        s = lax.axis_index("s")

        zero16 = jnp.zeros((16,), jnp.float32)
        one16 = jnp.ones((16,), jnp.float32)
        for j in range(128 // 16):
            ones_v[pl.ds(j * 16, 16)] = one16

        def _zrow(i, carry):
            for j in range(D // 16):
                rows_0[i, pl.ds(j * 16, 16)] = zero16
            return carry

        lax.fori_loop(0, C, _zrow, 0)

        def _zdeg(i, carry):
            dstage[pl.ds(i * 16, 16)] = zero16
            return carry

        lax.fori_loop(0, DEG_T // 16, _zdeg, 0)

        @pl.when(s < SROW_T)
        def _():
            for k in range(NW_FULL):
                pltpu.sync_copy(
                    rows_0.at[pl.ds(0, WCH)],
                    s_acc.at[pl.ds(s * ROWS_T + k * WCH, WCH)])
            if W_TAIL:
                pltpu.sync_copy(
                    rows_0.at[pl.ds(0, W_TAIL)],
                    s_acc.at[pl.ds(s * ROWS_T + NW_FULL * WCH, W_TAIL)])

        @pl.when(s < N // DEG_T)
        def _():
            pltpu.sync_copy(dstage, deg_acc.at[pl.ds(s * DEG_T, DEG_T)])

        plsc.subcore_barrier()

        g = c
        r = 1 - c

        def _gather(j, b, buf):
            pltpu.async_copy(nf_hbm.at[gidx.at[j]], buf, gsems[b])

        def _gwait(j, b, buf):
            pltpu.make_async_copy(nf_hbm.at[gidx.at[j]], buf, gsems[b]).wait()

        def _scat(j, b, buf):
            pltpu.async_copy(buf, s_acc.at[sidx.at[j]], ssems[b], add=True)
            pltpu.async_copy(
                ones_v.at[pl.ds(0, C)], deg_acc.at[sidx.at[j]], dsems[b],
                add=True)

        def _swait(j, b, buf):
            pltpu.make_async_copy(buf, s_acc.at[sidx.at[j]], ssems[b]).wait()
            pltpu.make_async_copy(
                ones_v.at[pl.ds(0, C)], deg_acc.at[sidx.at[j]], dsems[b]).wait()

        def _step(j, b, issue_next):
            if issue_next:
                p = (b + 2) % 3
                _swait(j - 1, p, rows[p])
                _gather(j + 2, p, rows[p])
            _gwait(j, b, rows[b])
            _scat(j, b, rows[b])

        def _block(blk, carry):
            pltpu.sync_copy(edges_hbm.at[g, s, blk], gidx)
            pltpu.sync_copy(edges_hbm.at[r, s, blk], sidx)

            _gather(0, 0, rows_0)
            _gather(1, 1, rows_1)
            _gather(2, 2, rows_2)
            _gwait(0, 0, rows_0)
            _scat(0, 0, rows_0)
            _step(jnp.int32(1), 1, True)

            def _body(jj, carry2):
                j = 3 * jj + 2
                _step(j, 2, True)
                _step(j + 1, 0, True)
                _step(j + 2, 1, True)
                return carry2

            lax.fori_loop(0, (BCH - 4) // 3, _body, 0)

            _step(jnp.int32(BCH - 2), (BCH - 2) % 3, False)
            _step(jnp.int32(BCH - 1), (BCH - 1) % 3, False)
            for j in (BCH - 3, BCH - 2, BCH - 1):
                _swait(j, j % 3, rows[j % 3])
            return carry

        lax.fori_loop(0, NBLK, _block, 0)

        plsc.subcore_barrier()

        @pl.when(s < SROW_T)
        def _():
            for k in range(NW_FULL):
                lo = s * ROWS_T + k * WCH
                pltpu.sync_copy(s_acc.at[pl.ds(lo, WCH)], rows_0.at[pl.ds(0, WCH)])
                pltpu.sync_copy(rows_0.at[pl.ds(0, WCH)], s_out.at[c, pl.ds(lo, WCH)])
            if W_TAIL:
                lo = s * ROWS_T + NW_FULL * WCH
                pltpu.sync_copy(
                    s_acc.at[pl.ds(lo, W_TAIL)], rows_1.at[pl.ds(0, W_TAIL)])
                pltpu.sync_copy(
                    rows_1.at[pl.ds(0, W_TAIL)], s_out.at[c, pl.ds(lo, W_TAIL)])

        @pl.when(s < N // DEG_T)
        def _():
            pltpu.sync_copy(deg_acc.at[pl.ds(s * DEG_T, DEG_T)], dstage)

            @pl.when(c == 0)
            def _():
                pltpu.sync_copy(dstage, deg_f_out.at[pl.ds(s * DEG_T, DEG_T)])

            @pl.when(c == 1)
            def _():
                pltpu.sync_copy(dstage, deg_r_out.at[pl.ds(s * DEG_T, DEG_T)])

    return sc_kernel


@functools.lru_cache(maxsize=None)
def _make_tc_combine(N, D, RPAD):
    R = 400
    G = N // R
    dn = (((1,), (1,)), ((), ()))
    f32 = jnp.float32

    def body(nf, sf, sr, df, dr, rp, wo, wi, ws, wr, bo, bi, bs, br,
             out, rout):
        i = pl.program_id(0)
        rp_v = rp[...]
        rw_o = lax.dot_general(rp_v, wo[...], dn, preferred_element_type=f32)
        rw_i = lax.dot_general(rp_v, wi[...], dn, preferred_element_type=f32)
        rw_s = lax.dot_general(rp_v, ws[...], dn, preferred_element_type=f32)
        c_f = bo[...] - rw_o[1:2, :]
        c_r = bi[...] - rw_i[2:3, :]
        c_s = bs[...] - rw_s[2:3, :]
        df_v = df[...]
        dr_v = dr[...]
        a_f = sf[...] * (1.0 / jnp.maximum(df_v, 1.0))
        a_r = sr[...] * (1.0 / jnp.maximum(dr_v, 1.0))
        acc = lax.dot_general(a_f, wo[...], dn, preferred_element_type=f32)
        acc += lax.dot_general(a_r, wi[...], dn, preferred_element_type=f32)
        acc += lax.dot_general(nf[...], ws[...], dn, preferred_element_type=f32)
        ind_f = jnp.where(df_v > 0.0, 1.0, 0.0)
        ind_r = jnp.where(dr_v > 0.0, 1.0, 0.0)
        out[...] = acc + ind_f * c_f + ind_r * c_r + c_s

        @pl.when(i == 0)
        def _():
            rout[...] = (
                lax.dot_general(rp_v, wr[...], dn, preferred_element_type=f32)
                + br[...]
            )

    row_blk = pl.BlockSpec((R, D), lambda i: (i, 0))
    col_blk = pl.BlockSpec((R, 1), lambda i: (i, 0))
    full = lambda shape: pl.BlockSpec(shape, lambda i: (0,) * len(shape))

    return pl.pallas_call(
        body,
        grid=(G,),
        in_specs=[
            row_blk, row_blk, row_blk, col_blk, col_blk,
            full((RPAD, D)),
            full((D, D)), full((D, D)), full((D, D)), full((D, D)),
            full((1, D)), full((1, D)), full((1, D)), full((1, D)),
        ],
        out_specs=[row_blk, full((RPAD, D))],
        out_shape=(
            jax.ShapeDtypeStruct((N, D), f32),
            jax.ShapeDtypeStruct((RPAD, D), f32),
        ),
    )


def kernel(n_feats, r_feats, edge_index, W_O_w, W_O_b, W_I_w, W_I_b,
           W_S_w, W_S_b, W_R_w, W_R_b):
    N, D = n_feats.shape
    E = edge_index.shape[1]
    NR = r_feats.shape[0]
    RPAD = 8

    NCH = (E // _NS) // _CHUNK
    edges_r = edge_index.reshape(2, _NS, 8, NCH // 8, _CHUNK)
    S, deg_f, deg_r = _make_sc_segment_sums(N, D, E)(n_feats, edges_r)

    rp = jnp.zeros((RPAD, D), jnp.float32).at[:NR].set(r_feats)
    n_out, r_out = _make_tc_combine(N, D, RPAD)(
        n_feats,
        S[0], S[1],
        deg_f.reshape(N, 1), deg_r.reshape(N, 1),
        rp,
        W_O_w, W_I_w, W_S_w, W_R_w,
        W_O_b.reshape(1, D), W_I_b.reshape(1, D),
        W_S_b.reshape(1, D), W_R_b.reshape(1, D),
    )
    return n_out, r_out[:NR]
